# baseline (speedup 1.0000x reference)
import functools

import jax
import jax.numpy as jnp
from jax import lax
from jax.experimental import pallas as pl
from jax.experimental.pallas import tpu as pltpu
from jax.experimental.pallas import tpu_sc as plsc

_NC = 2
_NS = 16
_NW = _NC * _NS

_L = 128


def _transpose_block(g, tb):
    rows = [lax.iota(jnp.int32, 16) + 16 * m for m in range(8)]
    for a in range(8):
        for s in range(8):
            col = jnp.full((16,), 8 * a + s, jnp.int32)
            for m in range(8):
                tb[a, s, pl.ds(16 * m, 16)] = plsc.load_gather(
                    g, [rows[m], col])


def _emb_body(n_blk, nj, ncb, table_hbm, idx_hbm, out_hbm,
              idx_sh, g0, g1, t0, t1, isem, gs0, gs1, os0, os1):
    wid = lax.axis_index("s") * _NC + lax.axis_index("c")
    blk0 = wid * n_blk

    pltpu.async_copy(idx_hbm.at[pl.ds(blk0, n_blk)], idx_sh, isem).wait()

    def fire_gather(t, g, gsem):
        return pltpu.async_copy(table_hbm.at[idx_sh.at[t]], g, gsem)

    def wait_gather(g, gsem):
        pltpu.make_async_copy(table_hbm.at[idx_sh.at[0]], g, gsem).wait()

    def out_slice(t):
        b = blk0 + t
        return out_hbm.at[b // ncb, :, b % ncb, :, :]

    def fire_write(t, tb, osem):
        pltpu.async_copy(tb, out_slice(t), osem)

    def wait_write(tb, osem):
        pltpu.make_async_copy(tb, out_slice(0), osem).wait()

    fire_gather(0, g0, gs0)

    def pair(k, carry):
        t = 2 * k
        wait_gather(g0, gs0)
        fire_gather(t + 1, g1, gs1)

        @pl.when(k > 0)
        def _():
            wait_write(t0, os0)
        _transpose_block(g0, t0)
        fire_write(t, t0, os0)

        wait_gather(g1, gs1)

        @pl.when(k < n_blk // 2 - 1)
        def _():
            fire_gather(t + 2, g0, gs0)

        @pl.when(k > 0)
        def _():
            wait_write(t1, os1)
        _transpose_block(g1, t1)
        fire_write(t + 1, t1, os1)
        return carry

    lax.fori_loop(0, n_blk // 2, pair, 0)
    wait_write(t0, os0)
    wait_write(t1, os1)


def kernel(x, weight):
    S0, S1 = x.shape
    B = S0 * S1
    D = weight.shape[1]
    ncb = S0 // _L
    n_blocks = S1 * ncb
    assert D == 64 and S0 % _L == 0 and n_blocks % (2 * _NW) == 0
    n_blk = n_blocks // _NW

    idx3 = x.T.astype(jnp.int32).reshape(n_blocks, _L)

    mesh = plsc.VectorSubcoreMesh(core_axis_name="c", subcore_axis_name="s")
    emb = functools.partial(
        pl.kernel,
        out_type=jax.ShapeDtypeStruct((S1, D // 8, ncb, 8, _L), jnp.float32),
        mesh=mesh,
        scratch_types=[
            pltpu.VMEM((n_blk, _L), jnp.int32),
            pltpu.VMEM((_L, D), jnp.float32),
            pltpu.VMEM((_L, D), jnp.float32),
            pltpu.VMEM((D // 8, 8, _L), jnp.float32),
            pltpu.VMEM((D // 8, 8, _L), jnp.float32),
            pltpu.SemaphoreType.DMA,
            pltpu.SemaphoreType.DMA,
            pltpu.SemaphoreType.DMA,
            pltpu.SemaphoreType.DMA,
            pltpu.SemaphoreType.DMA,
        ],
        compiler_params=pltpu.CompilerParams(use_tc_tiling_on_sc=False,
                                             needs_layout_passes=False),
    )(functools.partial(_emb_body, n_blk, S1, ncb))

    out5d = emb(weight, idx3)
    return out5d.transpose(2, 4, 0, 1, 3).reshape(S0, S1, D)


# speedup vs baseline: 1.6834x; 1.6834x over previous
import functools

import jax
import jax.numpy as jnp
from jax import lax
from jax.experimental import pallas as pl
from jax.experimental.pallas import tpu as pltpu
from jax.experimental.pallas import tpu_sc as plsc

_NC = 2
_NS = 16
_NW = _NC * _NS

_G = 128
_GPC = 4
_C = _G * _GPC
_NBUF = 2


def _emb_body(n_chunks, table_hbm, x_hbm, out_hbm,
              idx_all, rows0, rows1, isem, gsem0, gsem1, osem0, osem1):
    wid = lax.axis_index("s") * _NC + lax.axis_index("c")
    per_w = n_chunks * _C
    n_idx_rows = per_w // _G
    out_row0 = wid * per_w

    pltpu.async_copy(x_hbm.at[pl.ds(wid * n_idx_rows, n_idx_rows)],
                     idx_all, isem).wait()

    bufs = ((rows0, gsem0, osem0), (rows1, gsem1, osem1))

    def pair(k, carry):
        for b in range(_NBUF):
            t = _NBUF * k + b
            rows, gsem, osem = bufs[b]

            @pl.when(k > 0)
            def _wait_prev_write():
                pltpu.make_async_copy(rows, out_hbm.at[pl.ds(0, _C)],
                                      osem).wait()

            cps = [
                pltpu.async_copy(
                    table_hbm.at[idx_all.at[t * _GPC + j]],
                    rows.at[pl.ds(j * _G, _G)],
                    gsem,
                )
                for j in range(_GPC)
            ]
            for cp in cps:
                cp.wait()
            pltpu.async_copy(rows, out_hbm.at[pl.ds(out_row0 + t * _C, _C)],
                             osem)
        return carry

    lax.fori_loop(0, n_chunks // _NBUF, pair, 0)
    for rows, _, osem in bufs:
        pltpu.make_async_copy(rows, out_hbm.at[pl.ds(0, _C)], osem).wait()


def kernel(x, weight):
    S0, S1 = x.shape
    B = S0 * S1
    V, D = weight.shape
    assert B % (_NW * _C * _NBUF) == 0 and D == 64
    n_chunks = B // (_NW * _C)
    per_w = n_chunks * _C

    wp = jnp.pad(weight, ((0, 0), (0, 64))).reshape(2 * V, D)
    x2d = (x.astype(jnp.int32) * 2).reshape(B // _G, _G)

    mesh = plsc.VectorSubcoreMesh(core_axis_name="c", subcore_axis_name="s")
    emb = functools.partial(
        pl.kernel,
        out_type=jax.ShapeDtypeStruct((B, D), jnp.float32),
        mesh=mesh,
        scratch_types=[
            pltpu.VMEM((per_w // _G, _G), jnp.int32),
            pltpu.VMEM((_C, D), jnp.float32),
            pltpu.VMEM((_C, D), jnp.float32),
            pltpu.SemaphoreType.DMA,
            pltpu.SemaphoreType.DMA,
            pltpu.SemaphoreType.DMA,
            pltpu.SemaphoreType.DMA,
            pltpu.SemaphoreType.DMA,
        ],
        compiler_params=pltpu.CompilerParams(use_tc_tiling_on_sc=False),
    )(functools.partial(_emb_body, n_chunks))

    out = emb(wp, x2d)
    return out.reshape(S0, S1, D)


# speedup vs baseline: 2.3904x; 1.4200x over previous
import functools

import jax
import jax.numpy as jnp
from jax import lax
from jax.experimental import pallas as pl
from jax.experimental.pallas import tpu as pltpu
from jax.experimental.pallas import tpu_sc as plsc

_NC = 2
_NS = 16
_NW = _NC * _NS

_L = 128


_TP = 137


def _transpose_block(g, tb):
    d16 = [lax.iota(jnp.int32, 16) + 16 * k for k in range(4)]
    ia = [v // 8 for v in d16]
    is_ = [v % 8 for v in d16]

    @plsc.parallel_loop(0, 128, unroll=4)
    def _(l):
        lv = jnp.full((16,), 0, jnp.int32) + l
        for k in range(4):
            plsc.store_scatter(tb, [ia[k], is_[k], lv],
                               g[l, pl.ds(16 * k, 16)])


def _emb_body(n_blk, nj, ncb, table_hbm, idx_hbm, out_hbm,
              idx_sh, g0, g1, t0, t1, isem, gs0, gs1, os0, os1):
    wid = lax.axis_index("s") * _NC + lax.axis_index("c")
    blk0 = wid * n_blk

    pltpu.async_copy(idx_hbm.at[pl.ds(blk0, n_blk)], idx_sh, isem).wait()

    def fire_gather(t, g, gsem):
        return pltpu.async_copy(table_hbm.at[idx_sh.at[t]], g, gsem)

    def wait_gather(g, gsem):
        pltpu.make_async_copy(table_hbm.at[idx_sh.at[0]], g, gsem).wait()

    def out_slice(t):
        b = blk0 + t
        return out_hbm.at[b // ncb, :, b % ncb, :, :]

    def fire_write(t, tb, osem):
        pltpu.async_copy(tb.at[:, :, pl.ds(0, _L)], out_slice(t), osem)

    def wait_write(tb, osem):
        pltpu.make_async_copy(tb.at[:, :, pl.ds(0, _L)], out_slice(0),
                              osem).wait()

    fire_gather(0, g0, gs0)

    def pair(k, carry):
        t = 2 * k
        wait_gather(g0, gs0)
        fire_gather(t + 1, g1, gs1)

        @pl.when(k > 0)
        def _():
            wait_write(t0, os0)
        _transpose_block(g0, t0)
        fire_write(t, t0, os0)

        wait_gather(g1, gs1)

        @pl.when(k < n_blk // 2 - 1)
        def _():
            fire_gather(t + 2, g0, gs0)

        @pl.when(k > 0)
        def _():
            wait_write(t1, os1)
        _transpose_block(g1, t1)
        fire_write(t + 1, t1, os1)
        return carry

    lax.fori_loop(0, n_blk // 2, pair, 0)
    wait_write(t0, os0)
    wait_write(t1, os1)


def kernel(x, weight):
    S0, S1 = x.shape
    B = S0 * S1
    D = weight.shape[1]
    ncb = S0 // _L
    n_blocks = S1 * ncb
    assert D == 64 and S0 % _L == 0 and n_blocks % (2 * _NW) == 0
    n_blk = n_blocks // _NW

    idx3 = x.T.astype(jnp.int32).reshape(n_blocks, _L)

    mesh = plsc.VectorSubcoreMesh(core_axis_name="c", subcore_axis_name="s")
    emb = functools.partial(
        pl.kernel,
        out_type=jax.ShapeDtypeStruct((S1, D // 8, ncb, 8, _L), jnp.float32),
        mesh=mesh,
        scratch_types=[
            pltpu.VMEM((n_blk, _L), jnp.int32),
            pltpu.VMEM((_L, D), jnp.float32),
            pltpu.VMEM((_L, D), jnp.float32),
            pltpu.VMEM((D // 8, 8, _TP), jnp.float32),
            pltpu.VMEM((D // 8, 8, _TP), jnp.float32),
            pltpu.SemaphoreType.DMA,
            pltpu.SemaphoreType.DMA,
            pltpu.SemaphoreType.DMA,
            pltpu.SemaphoreType.DMA,
            pltpu.SemaphoreType.DMA,
        ],
        compiler_params=pltpu.CompilerParams(use_tc_tiling_on_sc=False,
                                             needs_layout_passes=False),
    )(functools.partial(_emb_body, n_blk, S1, ncb))

    out5d = emb(weight, idx3)
    return out5d.transpose(2, 4, 0, 1, 3).reshape(S0, S1, D)
